# baseline (speedup 1.0000x reference)
import dataclasses
import functools

import jax
import jax.numpy as jnp
from jax import lax
from jax.experimental import pallas as pl
from jax.experimental.pallas import tpu as pltpu
from jax.experimental.pallas import tpu_sc as plsc

N = 10000
E = 320000
F = 128
FP = 144
C_ONE = 128
C_S2 = 130
NEG_SLOPE = 0.05

NC = 2
NS = 16
NW = NC * NS
L = 16

B = 80
ER = E // B
EPW = E // NW
NB = EPW // B
AR = 25
RPT = 640
RLAST = N - (NS - 1) * RPT

B2R = 25
NB2 = NB // B2R

_mesh = plsc.VectorSubcoreMesh(core_axis_name="c", subcore_axis_name="s")

_cp = pltpu.CompilerParams()
if "needs_layout_passes" in pltpu.CompilerParams.__dataclass_fields__:
    _cp = dataclasses.replace(_cp, needs_layout_passes=False)
if "use_tc_tiling_on_sc" in pltpu.CompilerParams.__dataclass_fields__:
    _cp = dataclasses.replace(_cp, use_tc_tiling_on_sc=False)


@functools.partial(
    pl.kernel,
    out_type=[
        jax.ShapeDtypeStruct((ER, B), jnp.float32),
        jax.ShapeDtypeStruct((NC, N, FP), jnp.float32),
    ],
    mesh=_mesh,
    scratch_types=[
        pltpu.VMEM((N,), jnp.float32),
        pltpu.VMEM((2, 2, B), jnp.int32),
        pltpu.VMEM((B, FP), jnp.float32),
        pltpu.VMEM((B, FP), jnp.float32),
        pltpu.VMEM((2, AR, B), jnp.float32),
        pltpu.VMEM((2, B), jnp.int32),
        pltpu.VMEM_SHARED((N, FP), jnp.float32),
        pltpu.SemaphoreType.DMA,
        pltpu.SemaphoreType.DMA,
        pltpu.SemaphoreType.DMA,
        pltpu.SemaphoreType.DMA,
        pltpu.SemaphoreType.DMA,
        pltpu.SemaphoreType.DMA,
        pltpu.SemaphoreType.DMA,
    ],
    compiler_params=_cp,
)
def _edge_kernel(s1_hbm, sd_hbm, ht_hbm,
                 att_hbm, part_hbm,
                 s1_v, sd_v, hrow0_v, hrow1_v, att_v, sidx_v,
                 acc_sh, sem_i0, sem_i1, sem_g0, sem_g1, sem_s0, sem_s1,
                 sem_a):
    cid = lax.axis_index("c")
    sid = lax.axis_index("s")
    wid = sid * NC + cid

    zeros16 = jnp.zeros((L,), jnp.float32)
    lane_iota = lax.iota(jnp.int32, L)
    hrows = (hrow0_v, hrow1_v)
    sem_is = (sem_i0, sem_i1)
    sem_gs = (sem_g0, sem_g1)
    sem_ss = (sem_s0, sem_s1)

    @pl.loop(0, B)
    def _(r):
        @pl.loop(0, FP // L)
        def _(c):
            hrow0_v[r, pl.ds(c * L, L)] = zeros16

    @pl.when(sid < NS - 1)
    def _():
        @pl.loop(0, RPT // B)
        def _(j):
            pltpu.sync_copy(hrow0_v, acc_sh.at[pl.ds(sid * RPT + j * B, B)])

    @pl.when(sid == NS - 1)
    def _():
        @pl.loop(0, RLAST // B)
        def _(j):
            pltpu.sync_copy(hrow0_v,
                            acc_sh.at[pl.ds((NS - 1) * RPT + j * B, B)])

    pltpu.sync_copy(s1_hbm, s1_v)

    plsc.subcore_barrier()

    row0 = wid * NB

    def issue_idx(n, s):
        pltpu.async_copy(sd_hbm.at[row0 + n], sd_v.at[s], sem_is[s])

    def wait_idx(n, s):
        pltpu.make_async_copy(sd_hbm.at[row0 + n], sd_v.at[s],
                              sem_is[s]).wait()

    def issue_gather(s):
        pltpu.async_copy(ht_hbm.at[sd_v.at[s, 1]], hrows[s], sem_gs[s])

    def wait_gather(s):
        pltpu.make_async_copy(ht_hbm.at[sd_v.at[s, 1]], hrows[s],
                              sem_gs[s]).wait()

    def issue_scatter(s):
        pltpu.async_copy(hrows[s], acc_sh.at[sidx_v.at[s]], sem_ss[s],
                         add=True)

    def wait_scatter(s):
        pltpu.make_async_copy(hrows[s], acc_sh.at[sidx_v.at[s]],
                              sem_ss[s]).wait()

    def _att_desc(n):
        p = lax.rem(lax.div(n, AR), 2)
        return (att_v.at[p], att_hbm.at[pl.ds(row0 + n - (AR - 1), AR)])

    def issue_att_drain(n):
        a, b = _att_desc(n)
        pltpu.async_copy(a, b, sem_a)

    def wait_att_drain(n):
        a, b = _att_desc(n)
        pltpu.make_async_copy(a, b, sem_a).wait()

    def process(n, s, tail_wait):
        hrow_v = hrows[s]
        wait_gather(s)

        @pl.when(n + 1 < NB)
        def _():
            wait_idx(n + 1, 1 - s)
            if tail_wait:
                wait_scatter(1 - s)
            issue_gather(1 - s)

        arow = lax.rem(n, AR)
        p = lax.rem(lax.div(n, AR), 2)

        for k in range(B // L):
            s16 = sd_v[s, 0, pl.ds(k * L, L)]
            sidx_v[s, pl.ds(k * L, L)] = s16
            row16 = lane_iota + k * L
            e = (plsc.load_gather(s1_v, [s16])
                 + plsc.load_gather(hrow_v,
                                    [row16,
                                     jnp.zeros((L,), jnp.int32) + C_S2]))
            e = jnp.where(e >= 0.0, e, e * NEG_SLOPE)
            att_v[p, arow, pl.ds(k * L, L)] = jnp.exp(e)

        @pl.loop(0, B, step=4)
        def _(i):
            for u in range(4):
                ab = plsc.load_gather(att_v.at[p, arow],
                                      [jnp.zeros((L,), jnp.int32) + (i + u)])
                for c in range(FP // L):
                    hrow_v[i + u, pl.ds(c * L, L)] = (
                        hrow_v[i + u, pl.ds(c * L, L)] * ab)

        issue_scatter(s)

        if isinstance(n, int):
            if n % AR == AR - 1:
                if n >= 2 * AR - 1:
                    wait_att_drain(n - AR)
                issue_att_drain(n)
        else:
            @pl.when(arow == AR - 1)
            def _():
                @pl.when(n >= 2 * AR - 1)
                def _():
                    wait_att_drain(n - AR)
                issue_att_drain(n)

        @pl.when(n + 2 < NB)
        def _():
            issue_idx(n + 2, s)

    issue_idx(0, 0)
    issue_idx(1, 1)
    wait_idx(0, 0)
    issue_gather(0)

    process(0, 0, False)
    process(1, 1, True)

    @pl.loop(1, NB // 2)
    def _(t):
        process(2 * t, 0, True)
        process(2 * t + 1, 1, True)

    process(NB - 1, 0, True)
    wait_scatter(1)
    wait_scatter(0)
    wait_att_drain(NB - 1)

    plsc.subcore_barrier()

    @pl.when(sid < NS - 1)
    def _():
        pltpu.sync_copy(acc_sh.at[pl.ds(sid * RPT, RPT)],
                        part_hbm.at[cid, pl.ds(sid * RPT, RPT)])

    @pl.when(sid == NS - 1)
    def _():
        pltpu.sync_copy(acc_sh.at[pl.ds((NS - 1) * RPT, RLAST)],
                        part_hbm.at[cid, pl.ds((NS - 1) * RPT, RLAST)])


@functools.partial(
    pl.kernel,
    out_type=jax.ShapeDtypeStruct((ER, B), jnp.float32),
    mesh=_mesh,
    scratch_types=[
        pltpu.VMEM((N,), jnp.float32),
        pltpu.VMEM((2, B2R, B), jnp.int32),
        pltpu.VMEM((2, B2R, B), jnp.float32),
        pltpu.VMEM((2, B2R, B), jnp.float32),
        pltpu.SemaphoreType.DMA,
        pltpu.SemaphoreType.DMA,
        pltpu.SemaphoreType.DMA,
        pltpu.SemaphoreType.DMA,
        pltpu.SemaphoreType.DMA,
    ],
    compiler_params=_cp,
)
def _alpha_kernel(rs_hbm, src_hbm, att_hbm, alpha_hbm,
                  rs_v, src_v, att_v, al_v,
                  sem_r, sem_l0, sem_l1, sem_t0, sem_t1):
    cid = lax.axis_index("c")
    sid = lax.axis_index("s")
    wid = sid * NC + cid
    row0 = wid * NB
    sem_ls = (sem_l0, sem_l1)
    sem_ts = (sem_t0, sem_t1)

    def grow(c):
        return row0 + c * B2R

    def issue_load(c, s):
        pltpu.async_copy(src_hbm.at[pl.ds(grow(c), B2R)], src_v.at[s],
                         sem_ls[s])
        pltpu.async_copy(att_hbm.at[pl.ds(grow(c), B2R)], att_v.at[s],
                         sem_ls[s])

    def wait_load(c, s):
        pltpu.make_async_copy(src_hbm.at[pl.ds(grow(c), B2R)], src_v.at[s],
                              sem_ls[s]).wait()
        pltpu.make_async_copy(att_hbm.at[pl.ds(grow(c), B2R)], att_v.at[s],
                              sem_ls[s]).wait()

    def issue_store(c, s):
        pltpu.async_copy(al_v.at[s], alpha_hbm.at[pl.ds(grow(c), B2R)],
                         sem_ts[s])

    def wait_store(c, s):
        pltpu.make_async_copy(al_v.at[s], alpha_hbm.at[pl.ds(grow(c), B2R)],
                              sem_ts[s]).wait()

    pltpu.async_copy(rs_hbm, rs_v, sem_r)
    issue_load(0, 0)
    issue_load(1, 1)
    pltpu.make_async_copy(rs_hbm, rs_v, sem_r).wait()

    for c in range(NB2):
        s = c % 2
        wait_load(c, s)
        if c >= 2:
            wait_store(c - 2, s)

        @pl.loop(0, B2R)
        def _(r):
            for k in range(B // L):
                s16 = src_v[s, r, pl.ds(k * L, L)]
                r16 = plsc.load_gather(rs_v, [s16])
                al_v[s, r, pl.ds(k * L, L)] = (
                    att_v[s, r, pl.ds(k * L, L)] / r16)

        issue_store(c, s)
        if c + 2 < NB2:
            issue_load(c + 2, s)

    wait_store(NB2 - 2, (NB2 - 2) % 2)
    wait_store(NB2 - 1, (NB2 - 1) % 2)


_R0 = 2000


def _tc_h_body(x_ref, w_ref, a_ref, b_ref, ht_ref, s_ref):
    h = jnp.dot(x_ref[...], w_ref[...],
                preferred_element_type=jnp.float32)
    s12 = jnp.dot(h, a_ref[...],
                  preferred_element_type=jnp.float32) + b_ref[...]
    ones = jnp.ones((_R0, 1), jnp.float32)
    pad = jnp.zeros((_R0, FP - F - 3), jnp.float32)
    ht_ref[...] = jnp.concatenate([h, ones, s12, pad], axis=1)
    s_ref[...] = s12


def _tc_h(x, W, acat, b2):
    return pl.pallas_call(
        _tc_h_body,
        grid=(N // _R0,),
        in_specs=[
            pl.BlockSpec((_R0, F), lambda i: (i, 0)),
            pl.BlockSpec((F, F), lambda i: (0, 0)),
            pl.BlockSpec((F, 2), lambda i: (0, 0)),
            pl.BlockSpec((1, 2), lambda i: (0, 0)),
        ],
        out_specs=[
            pl.BlockSpec((_R0, FP), lambda i: (i, 0)),
            pl.BlockSpec((_R0, 2), lambda i: (i, 0)),
        ],
        out_shape=[
            jax.ShapeDtypeStruct((N, FP), jnp.float32),
            jax.ShapeDtypeStruct((N, 2), jnp.float32),
        ],
    )(x, W, acat, b2)


def _tc_norm_body(p_ref, out_ref, rs_ref):
    s = p_ref[0] + p_ref[1]
    rs = s[:, C_ONE]
    inv = jnp.where(rs > 0.0, 1.0 / rs, 0.0)
    out_ref[...] = s[:, :F] * inv[:, None]
    rs_ref[...] = rs[:, None]


def _tc_norm(part):
    return pl.pallas_call(
        _tc_norm_body,
        grid=(N // _R0,),
        in_specs=[pl.BlockSpec((2, _R0, FP), lambda i: (0, i, 0))],
        out_specs=[
            pl.BlockSpec((_R0, F), lambda i: (i, 0)),
            pl.BlockSpec((_R0, 1), lambda i: (i, 0)),
        ],
        out_shape=[
            jax.ShapeDtypeStruct((N, F), jnp.float32),
            jax.ShapeDtypeStruct((N, 1), jnp.float32),
        ],
    )(part)


def kernel(x, edge_index, W, a_w, a_b):
    acat = jnp.concatenate([a_w[:F, :], a_w[F:, :]], axis=1)
    b2 = jnp.stack([a_b[0], jnp.zeros((), jnp.float32)]).reshape(1, 2)

    ht, s12 = _tc_h(x, W, acat, b2)
    s1 = s12[:, 0]

    src2 = edge_index[0].reshape(ER, B)
    dst2 = edge_index[1].reshape(ER, B)
    sd = jnp.stack([src2, dst2], axis=1)

    att2, part = _edge_kernel(s1, sd, ht)
    out, rs2 = _tc_norm(part)
    rs = rs2.reshape(N)
    alpha2 = _alpha_kernel(rs, src2, att2)
    return (out, alpha2.reshape(E))


# speedup vs baseline: 1.0590x; 1.0590x over previous
import dataclasses
import functools

import jax
import jax.numpy as jnp
from jax import lax
from jax.experimental import pallas as pl
from jax.experimental.pallas import tpu as pltpu
from jax.experimental.pallas import tpu_sc as plsc

N = 10000
E = 320000
F = 128
FP = 144
C_ONE = 128
C_S2 = 130
NEG_SLOPE = 0.05

NC = 2
NS = 16
NW = NC * NS
L = 16

B = 80
ER = E // B
EPW = E // NW
NB = EPW // B
AR = 25
RPT = 640
RLAST = N - (NS - 1) * RPT

B2R = 25
NB2 = NB // B2R

_mesh = plsc.VectorSubcoreMesh(core_axis_name="c", subcore_axis_name="s")

_cp = pltpu.CompilerParams()
if "needs_layout_passes" in pltpu.CompilerParams.__dataclass_fields__:
    _cp = dataclasses.replace(_cp, needs_layout_passes=False)
if "use_tc_tiling_on_sc" in pltpu.CompilerParams.__dataclass_fields__:
    _cp = dataclasses.replace(_cp, use_tc_tiling_on_sc=False)


@functools.partial(
    pl.kernel,
    out_type=[
        jax.ShapeDtypeStruct((ER, B), jnp.float32),
        jax.ShapeDtypeStruct((NC, N, FP), jnp.float32),
    ],
    mesh=_mesh,
    scratch_types=[
        pltpu.VMEM((N,), jnp.float32),
        pltpu.VMEM((2, B), jnp.int32),
        pltpu.VMEM((2, B), jnp.int32),
        pltpu.VMEM((B, FP), jnp.float32),
        pltpu.VMEM((B, FP), jnp.float32),
        pltpu.VMEM((AR, B), jnp.float32),
        pltpu.VMEM((2, B), jnp.int32),
        pltpu.VMEM_SHARED((N, FP), jnp.float32),
        pltpu.SemaphoreType.DMA,
        pltpu.SemaphoreType.DMA,
        pltpu.SemaphoreType.DMA,
        pltpu.SemaphoreType.DMA,
        pltpu.SemaphoreType.DMA,
        pltpu.SemaphoreType.DMA,
    ],
    compiler_params=_cp,
)
def _edge_kernel(s1_hbm, src_hbm, dst_hbm, ht_hbm,
                 att_hbm, part_hbm,
                 s1_v, src_v, dst_v, hrow0_v, hrow1_v, att_v, sidx_v,
                 acc_sh, sem_i0, sem_i1, sem_g0, sem_g1, sem_s0, sem_s1):
    cid = lax.axis_index("c")
    sid = lax.axis_index("s")
    wid = sid * NC + cid

    zeros16 = jnp.zeros((L,), jnp.float32)
    lane_iota = lax.iota(jnp.int32, L)
    hrows = (hrow0_v, hrow1_v)
    sem_is = (sem_i0, sem_i1)
    sem_gs = (sem_g0, sem_g1)
    sem_ss = (sem_s0, sem_s1)

    @pl.loop(0, B)
    def _(r):
        @pl.loop(0, FP // L)
        def _(c):
            hrow0_v[r, pl.ds(c * L, L)] = zeros16

    @pl.when(sid < NS - 1)
    def _():
        @pl.loop(0, RPT // B)
        def _(j):
            pltpu.sync_copy(hrow0_v, acc_sh.at[pl.ds(sid * RPT + j * B, B)])

    @pl.when(sid == NS - 1)
    def _():
        @pl.loop(0, RLAST // B)
        def _(j):
            pltpu.sync_copy(hrow0_v,
                            acc_sh.at[pl.ds((NS - 1) * RPT + j * B, B)])

    pltpu.sync_copy(s1_hbm, s1_v)

    plsc.subcore_barrier()

    row0 = wid * NB

    def issue_idx(n, s):
        pltpu.async_copy(src_hbm.at[row0 + n], src_v.at[s], sem_is[s])
        pltpu.async_copy(dst_hbm.at[row0 + n], dst_v.at[s], sem_is[s])

    def wait_idx(n, s):
        pltpu.make_async_copy(src_hbm.at[row0 + n], src_v.at[s],
                              sem_is[s]).wait()
        pltpu.make_async_copy(dst_hbm.at[row0 + n], dst_v.at[s],
                              sem_is[s]).wait()

    def issue_gather(s):
        pltpu.async_copy(ht_hbm.at[dst_v.at[s]], hrows[s], sem_gs[s])

    def wait_gather(s):
        pltpu.make_async_copy(ht_hbm.at[dst_v.at[s]], hrows[s],
                              sem_gs[s]).wait()

    def issue_scatter(s):
        pltpu.async_copy(hrows[s], acc_sh.at[sidx_v.at[s]], sem_ss[s],
                         add=True)

    def wait_scatter(s):
        pltpu.make_async_copy(hrows[s], acc_sh.at[sidx_v.at[s]],
                              sem_ss[s]).wait()

    def process(n, s, tail_wait):
        hrow_v = hrows[s]
        wait_gather(s)

        @pl.when(n + 1 < NB)
        def _():
            wait_idx(n + 1, 1 - s)
            if tail_wait:
                wait_scatter(1 - s)
            issue_gather(1 - s)

        arow = lax.rem(n, AR)

        for k in range(B // L):
            s16 = src_v[s, pl.ds(k * L, L)]
            sidx_v[s, pl.ds(k * L, L)] = s16
            row16 = lane_iota + k * L
            e = (plsc.load_gather(s1_v, [s16])
                 + plsc.load_gather(hrow_v,
                                    [row16,
                                     jnp.zeros((L,), jnp.int32) + C_S2]))
            e = jnp.where(e >= 0.0, e, e * NEG_SLOPE)
            att_v[arow, pl.ds(k * L, L)] = jnp.exp(e)

        @pl.loop(0, B, step=4)
        def _(i):
            for u in range(4):
                ab = plsc.load_gather(att_v.at[arow],
                                      [jnp.zeros((L,), jnp.int32) + (i + u)])
                for c in range(FP // L):
                    hrow_v[i + u, pl.ds(c * L, L)] = (
                        hrow_v[i + u, pl.ds(c * L, L)] * ab)

        issue_scatter(s)

        @pl.when(arow == AR - 1)
        def _():
            pltpu.sync_copy(att_v, att_hbm.at[pl.ds(row0 + n - (AR - 1), AR)])

        @pl.when(n + 2 < NB)
        def _():
            issue_idx(n + 2, s)

    issue_idx(0, 0)
    issue_idx(1, 1)
    wait_idx(0, 0)
    issue_gather(0)

    process(0, 0, False)
    process(1, 1, True)

    @pl.loop(1, NB // 2)
    def _(t):
        process(2 * t, 0, True)
        process(2 * t + 1, 1, True)

    process(NB - 1, 0, True)
    wait_scatter(1)
    wait_scatter(0)

    plsc.subcore_barrier()

    @pl.when(sid < NS - 1)
    def _():
        pltpu.sync_copy(acc_sh.at[pl.ds(sid * RPT, RPT)],
                        part_hbm.at[cid, pl.ds(sid * RPT, RPT)])

    @pl.when(sid == NS - 1)
    def _():
        pltpu.sync_copy(acc_sh.at[pl.ds((NS - 1) * RPT, RLAST)],
                        part_hbm.at[cid, pl.ds((NS - 1) * RPT, RLAST)])


@functools.partial(
    pl.kernel,
    out_type=jax.ShapeDtypeStruct((ER, B), jnp.float32),
    mesh=_mesh,
    scratch_types=[
        pltpu.VMEM((N,), jnp.float32),
        pltpu.VMEM((2, B2R, B), jnp.int32),
        pltpu.VMEM((2, B2R, B), jnp.float32),
        pltpu.VMEM((2, B2R, B), jnp.float32),
        pltpu.SemaphoreType.DMA,
        pltpu.SemaphoreType.DMA,
        pltpu.SemaphoreType.DMA,
        pltpu.SemaphoreType.DMA,
        pltpu.SemaphoreType.DMA,
    ],
    compiler_params=_cp,
)
def _alpha_kernel(rs_hbm, src_hbm, att_hbm, alpha_hbm,
                  rs_v, src_v, att_v, al_v,
                  sem_r, sem_l0, sem_l1, sem_t0, sem_t1):
    cid = lax.axis_index("c")
    sid = lax.axis_index("s")
    wid = sid * NC + cid
    row0 = wid * NB
    sem_ls = (sem_l0, sem_l1)
    sem_ts = (sem_t0, sem_t1)

    def grow(c):
        return row0 + c * B2R

    def issue_load(c, s):
        pltpu.async_copy(src_hbm.at[pl.ds(grow(c), B2R)], src_v.at[s],
                         sem_ls[s])
        pltpu.async_copy(att_hbm.at[pl.ds(grow(c), B2R)], att_v.at[s],
                         sem_ls[s])

    def wait_load(c, s):
        pltpu.make_async_copy(src_hbm.at[pl.ds(grow(c), B2R)], src_v.at[s],
                              sem_ls[s]).wait()
        pltpu.make_async_copy(att_hbm.at[pl.ds(grow(c), B2R)], att_v.at[s],
                              sem_ls[s]).wait()

    def issue_store(c, s):
        pltpu.async_copy(al_v.at[s], alpha_hbm.at[pl.ds(grow(c), B2R)],
                         sem_ts[s])

    def wait_store(c, s):
        pltpu.make_async_copy(al_v.at[s], alpha_hbm.at[pl.ds(grow(c), B2R)],
                              sem_ts[s]).wait()

    pltpu.async_copy(rs_hbm, rs_v, sem_r)
    issue_load(0, 0)
    issue_load(1, 1)
    pltpu.make_async_copy(rs_hbm, rs_v, sem_r).wait()

    for c in range(NB2):
        s = c % 2
        wait_load(c, s)
        if c >= 2:
            wait_store(c - 2, s)

        @pl.loop(0, B2R)
        def _(r):
            for k in range(B // L):
                s16 = src_v[s, r, pl.ds(k * L, L)]
                r16 = plsc.load_gather(rs_v, [s16])
                al_v[s, r, pl.ds(k * L, L)] = (
                    att_v[s, r, pl.ds(k * L, L)] / r16)

        issue_store(c, s)
        if c + 2 < NB2:
            issue_load(c + 2, s)

    wait_store(NB2 - 2, (NB2 - 2) % 2)
    wait_store(NB2 - 1, (NB2 - 1) % 2)


_R0 = 2000


def _tc_h_body(x_ref, w_ref, a_ref, b_ref, ht_ref, s_ref):
    h = jnp.dot(x_ref[...], w_ref[...],
                preferred_element_type=jnp.float32)
    s12 = jnp.dot(h, a_ref[...],
                  preferred_element_type=jnp.float32) + b_ref[...]
    ones = jnp.ones((_R0, 1), jnp.float32)
    pad = jnp.zeros((_R0, FP - F - 3), jnp.float32)
    ht_ref[...] = jnp.concatenate([h, ones, s12, pad], axis=1)
    s_ref[...] = s12


def _tc_h(x, W, acat, b2):
    return pl.pallas_call(
        _tc_h_body,
        grid=(N // _R0,),
        in_specs=[
            pl.BlockSpec((_R0, F), lambda i: (i, 0)),
            pl.BlockSpec((F, F), lambda i: (0, 0)),
            pl.BlockSpec((F, 2), lambda i: (0, 0)),
            pl.BlockSpec((1, 2), lambda i: (0, 0)),
        ],
        out_specs=[
            pl.BlockSpec((_R0, FP), lambda i: (i, 0)),
            pl.BlockSpec((_R0, 2), lambda i: (i, 0)),
        ],
        out_shape=[
            jax.ShapeDtypeStruct((N, FP), jnp.float32),
            jax.ShapeDtypeStruct((N, 2), jnp.float32),
        ],
    )(x, W, acat, b2)


def _tc_norm_body(p_ref, out_ref, rs_ref):
    s = p_ref[0] + p_ref[1]
    rs = s[:, C_ONE]
    inv = jnp.where(rs > 0.0, 1.0 / rs, 0.0)
    out_ref[...] = s[:, :F] * inv[:, None]
    rs_ref[...] = rs[:, None]


def _tc_norm(part):
    return pl.pallas_call(
        _tc_norm_body,
        grid=(N // _R0,),
        in_specs=[pl.BlockSpec((2, _R0, FP), lambda i: (0, i, 0))],
        out_specs=[
            pl.BlockSpec((_R0, F), lambda i: (i, 0)),
            pl.BlockSpec((_R0, 1), lambda i: (i, 0)),
        ],
        out_shape=[
            jax.ShapeDtypeStruct((N, F), jnp.float32),
            jax.ShapeDtypeStruct((N, 1), jnp.float32),
        ],
    )(part)


def kernel(x, edge_index, W, a_w, a_b):
    acat = jnp.concatenate([a_w[:F, :], a_w[F:, :]], axis=1)
    b2 = jnp.stack([a_b[0], jnp.zeros((), jnp.float32)]).reshape(1, 2)

    ht, s12 = _tc_h(x, W, acat, b2)
    s1 = s12[:, 0]

    src2 = edge_index[0].reshape(ER, B)
    dst2 = edge_index[1].reshape(ER, B)

    att2, part = _edge_kernel(s1, src2, dst2, ht)
    out, rs2 = _tc_norm(part)
    rs = rs2.reshape(N)
    alpha2 = _alpha_kernel(rs, src2, att2)
    return (out, alpha2.reshape(E))
